# baseline (speedup 1.0000x reference)
import jax
import jax.numpy as jnp
from jax.experimental import pallas as pl
from jax.experimental.pallas import tpu as pltpu

_N = 4096
_L = 2
_BLK = 256


def _body(a0_ref, a1_ref, o_ref):
    o_ref[...] = a0_ref[0:8, 0:128] + a1_ref[0:8, 0:128]


def kernel(features, adj_list, W_gcn, b_gcn, W_w, W_y, b_y, sparse):
    a0 = adj_list[0, 0]
    a1 = adj_list[1, 0]
    nb = _N // _BLK
    o = pl.pallas_call(
        _body,
        grid=(nb,),
        in_specs=[pl.BlockSpec((_BLK, _N), lambda b: (b, 0)),
                  pl.BlockSpec((_BLK, _N), lambda b: (b, 0))],
        out_specs=pl.BlockSpec((8, 128), lambda b: (0, 0)),
        out_shape=jax.ShapeDtypeStruct((8, 128), jnp.float32),
    )(a0, a1)
    return o


# speedup vs baseline: 1.5417x; 1.5417x over previous
import jax
import jax.numpy as jnp
from jax.experimental import pallas as pl
from jax.experimental.pallas import tpu as pltpu

_N = 4096
_IN = 512
_H = 512
_L = 2
_BLK = 128


def _body(f_ref, wg_ref, ww_ref, wy_ref, bg_ref, by_ref,
          adj_ref, final_ref, layers_ref, seq_s, v_s):
    @pl.when(pl.program_id(0) == 0)
    def _prologue():
        f = f_ref[...].astype(jnp.bfloat16)
        for l in range(_L):
            wg = wg_ref[l].astype(jnp.bfloat16)
            seq_s[l] = jax.lax.dot_general(
                f, wg, (((1,), (1,)), ((), ())),
                preferred_element_type=jnp.float32).astype(jnp.bfloat16)
            ww = ww_ref[l]
            wy = wy_ref[l, 0]
            v_s[l, 0:1, :] = jnp.sum(ww * wy[:, None], axis=0)[None]

    embs = []
    for l in range(_L):
        a = adj_ref[l].astype(jnp.bfloat16)
        e = jax.lax.dot_general(
            a, seq_s[l], (((1,), (0,)), ((), ())),
            preferred_element_type=jnp.float32)
        e = jnp.maximum(e + bg_ref[l, 0], 0.0)
        layers_ref[l] = e
        embs.append(e)
    ws = []
    for l in range(_L):
        v = v_s[l, 0]
        s = jnp.sum(embs[l] * v, axis=1, keepdims=True) + by_ref[0, l]
        ws.append(jnp.exp(jnp.tanh(s)))
    inv = 1.0 / (ws[0] + ws[1])
    final_ref[...] = (ws[0] * embs[0] + ws[1] * embs[1]) * inv


def kernel(features, adj_list, W_gcn, b_gcn, W_w, W_y, b_y, sparse):
    f = features[0]
    adj = adj_list[:, 0]
    wy3 = W_y.reshape(_L, 1, _H)
    bg3 = b_gcn.reshape(_L, 1, _H)
    by2 = b_y.reshape(1, _L)

    nb = _N // _BLK
    final, layers = pl.pallas_call(
        _body,
        grid=(nb,),
        in_specs=[
            pl.BlockSpec((_N, _IN), lambda b: (0, 0)),
            pl.BlockSpec((_L, _H, _IN), lambda b: (0, 0, 0)),
            pl.BlockSpec((_L, _H, _H), lambda b: (0, 0, 0)),
            pl.BlockSpec((_L, 1, _H), lambda b: (0, 0, 0)),
            pl.BlockSpec((_L, 1, _H), lambda b: (0, 0, 0)),
            pl.BlockSpec((1, _L), lambda b: (0, 0)),
            pl.BlockSpec((_L, _BLK, _N), lambda b: (0, b, 0)),
        ],
        out_specs=[
            pl.BlockSpec((_BLK, _H), lambda b: (b, 0)),
            pl.BlockSpec((_L, _BLK, _H), lambda b: (0, b, 0)),
        ],
        out_shape=[
            jax.ShapeDtypeStruct((_N, _H), jnp.float32),
            jax.ShapeDtypeStruct((_L, _N, _H), jnp.float32),
        ],
        scratch_shapes=[
            pltpu.VMEM((_L, _N, _H), jnp.bfloat16),
            pltpu.VMEM((_L, 8, _H), jnp.float32),
        ],
    )(f, W_gcn, W_w, wy3, bg3, by2, adj)

    return (final, layers)


# speedup vs baseline: 2.0670x; 1.3407x over previous
import jax
import jax.numpy as jnp
from jax.experimental import pallas as pl
from jax.experimental.pallas import tpu as pltpu

_N = 4096
_IN = 512
_H = 512
_L = 2
_BLK = 256
_NB = _N // _BLK
_K = 3


def _adj_cp(adj_hbm, abufs, sems, b):
    return pltpu.make_async_copy(
        adj_hbm.at[:, pl.ds(b * _BLK, _BLK), :], abufs[b % _K], sems[b % _K])


def _body(wg_ref, ww_ref, wy_ref, bg_ref, by_ref,
          f_hbm, adj_hbm, final_hbm, layers_hbm,
          fbuf, seq_s, a0, a1, a2, of0, of1, ol0, ol1,
          fsem, as0, as1, as2, ofs0, ofs1, ols0, ols1):
    abufs = (a0, a1, a2)
    asems = (as0, as1, as2)
    ofb, ofs = (of0, of1), (ofs0, ofs1)
    olb, ols = (ol0, ol1), (ols0, ols1)

    fcp = pltpu.make_async_copy(f_hbm, fbuf, fsem)
    fcp.start()
    for k in range(_K):
        _adj_cp(adj_hbm, abufs, asems, k).start()

    fcp.wait()
    f16 = fbuf[...].astype(jnp.bfloat16)
    vs = []
    for l in range(_L):
        wg = wg_ref[l].astype(jnp.bfloat16)
        seq_s[l] = jax.lax.dot_general(
            f16, wg, (((1,), (1,)), ((), ())),
            preferred_element_type=jnp.float32).astype(jnp.bfloat16)
        vs.append(jnp.sum(ww_ref[l] * wy_ref[l, 0][:, None], axis=0))

    for b in range(_NB):
        k = b % _K
        _adj_cp(adj_hbm, abufs, asems, b).wait()
        embs = []
        for l in range(_L):
            a = abufs[k][l].astype(jnp.bfloat16)
            e = jax.lax.dot_general(
                a, seq_s[l], (((1,), (0,)), ((), ())),
                preferred_element_type=jnp.float32)
            embs.append(jnp.maximum(e + bg_ref[l, 0], 0.0))
        if b + _K < _NB:
            _adj_cp(adj_hbm, abufs, asems, b + _K).start()
        ws = []
        for l in range(_L):
            s = jnp.sum(embs[l] * vs[l], axis=1, keepdims=True) + by_ref[0, l]
            ws.append(jnp.exp(jnp.tanh(s)))
        inv = 1.0 / (ws[0] + ws[1])
        s2 = b % 2
        if b >= 2:
            pltpu.make_async_copy(
                ofb[s2], final_hbm.at[pl.ds((b - 2) * _BLK, _BLK), :],
                ofs[s2]).wait()
            pltpu.make_async_copy(
                olb[s2], layers_hbm.at[:, pl.ds((b - 2) * _BLK, _BLK), :],
                ols[s2]).wait()
        ofb[s2][...] = (ws[0] * embs[0] + ws[1] * embs[1]) * inv
        for l in range(_L):
            olb[s2][l] = embs[l]
        pltpu.make_async_copy(
            ofb[s2], final_hbm.at[pl.ds(b * _BLK, _BLK), :], ofs[s2]).start()
        pltpu.make_async_copy(
            olb[s2], layers_hbm.at[:, pl.ds(b * _BLK, _BLK), :], ols[s2]).start()

    for b in (_NB - 2, _NB - 1):
        s2 = b % 2
        pltpu.make_async_copy(
            ofb[s2], final_hbm.at[pl.ds(b * _BLK, _BLK), :], ofs[s2]).wait()
        pltpu.make_async_copy(
            olb[s2], layers_hbm.at[:, pl.ds(b * _BLK, _BLK), :], ols[s2]).wait()


def kernel(features, adj_list, W_gcn, b_gcn, W_w, W_y, b_y, sparse):
    f = features[0]
    adj = adj_list[:, 0]
    wy3 = W_y.reshape(_L, 1, _H)
    bg3 = b_gcn.reshape(_L, 1, _H)
    by2 = b_y.reshape(1, _L)

    vmem = pl.BlockSpec(memory_space=pltpu.MemorySpace.VMEM)
    hbm = pl.BlockSpec(memory_space=pltpu.MemorySpace.HBM)
    final, layers = pl.pallas_call(
        _body,
        in_specs=[vmem, vmem, vmem, vmem, vmem, hbm, hbm],
        out_specs=[hbm, hbm],
        out_shape=[
            jax.ShapeDtypeStruct((_N, _H), jnp.float32),
            jax.ShapeDtypeStruct((_L, _N, _H), jnp.float32),
        ],
        scratch_shapes=[
            pltpu.VMEM((_N, _IN), jnp.float32),
            pltpu.VMEM((_L, _N, _H), jnp.bfloat16),
            pltpu.VMEM((_L, _BLK, _N), jnp.float32),
            pltpu.VMEM((_L, _BLK, _N), jnp.float32),
            pltpu.VMEM((_L, _BLK, _N), jnp.float32),
            pltpu.VMEM((_BLK, _H), jnp.float32),
            pltpu.VMEM((_BLK, _H), jnp.float32),
            pltpu.VMEM((_L, _BLK, _H), jnp.float32),
            pltpu.VMEM((_L, _BLK, _H), jnp.float32),
            pltpu.SemaphoreType.DMA,
            pltpu.SemaphoreType.DMA,
            pltpu.SemaphoreType.DMA,
            pltpu.SemaphoreType.DMA,
            pltpu.SemaphoreType.DMA,
            pltpu.SemaphoreType.DMA,
            pltpu.SemaphoreType.DMA,
            pltpu.SemaphoreType.DMA,
        ],
    )(W_gcn, W_w, wy3, bg3, by2, f, adj)

    return (final, layers)
